# baseline (speedup 1.0000x reference)
import functools

import jax
import jax.numpy as jnp
from jax import lax
from jax.experimental import pallas as pl
from jax.experimental.pallas import tpu as pltpu
from jax.experimental.pallas import tpu_sc as plsc

RADIUS2 = 0.2 * 0.2
NSMP = 32
L = 16
NW = 32


@functools.lru_cache(maxsize=None)
def _make_fused(B, N, S, C):
    NB = NW // B
    SQ = S // NB
    NCH = N // L
    BPC = B // 2
    CPT = (B * C) // NW
    CH = 128
    NQC = S // CH
    KPC = CH * NSMP // L
    OC = C + 3
    mesh = plsc.VectorSubcoreMesh(core_axis_name="c", subcore_axis_name="s")

    @functools.partial(
        pl.kernel,
        out_type=jax.ShapeDtypeStruct((B * OC, S * NSMP), jnp.float32),
        mesh=mesh,
        compiler_params=pltpu.CompilerParams(needs_layout_passes=False,
                                             use_tc_tiling_on_sc=False),
        scratch_types=[
            pltpu.VMEM((3 * SQ,), jnp.float32),
            pltpu.VMEM((64,), jnp.int32),
            pltpu.VMEM((CH * NSMP,), jnp.int32),
            pltpu.VMEM_SHARED((BPC * S * NSMP,), jnp.int32),
        ],
    )
    def fused(xyz_hbm, q_hbm, feat_hbm, out_hbm, qrs, idxbuf, idxc, idx_sh):
        wid = lax.axis_index("c") * 16 + lax.axis_index("s")

        b = wid // NB
        q0 = (wid % NB) * SQ
        lb = b % BPC

        def phase_a(pts, idx_out, gx_out):
            for c in range(3):
                pltpu.sync_copy(xyz_hbm.at[pl.ds((b * 3 + c) * N, N)],
                                pts.at[pl.ds(c * N, N)])
                pltpu.sync_copy(q_hbm.at[pl.ds((b * 3 + c) * S + q0, SQ)],
                                qrs.at[pl.ds(c * SQ, SQ)])
            iota = lax.broadcasted_iota(jnp.int32, (L,), 0)
            zeros16 = jnp.zeros((L,), jnp.int32)

            def per_query(s, carry):
                idxbuf[pl.ds(0, L)] = zeros16
                qx = plsc.load_gather(qrs, [jnp.full((L,), s, jnp.int32)])
                qy = plsc.load_gather(qrs, [jnp.full((L,), SQ + s, jnp.int32)])
                qz = plsc.load_gather(qrs,
                                      [jnp.full((L,), 2 * SQ + s, jnp.int32)])

                def cond(cr):
                    ch, cnt = cr
                    return jnp.logical_and(cnt < NSMP, ch < NCH // 2)

                def body(cr):
                    ch, cnt = cr
                    i0 = ch * (2 * L)
                    cnt1 = cnt
                    for h in range(2):
                        ib = i0 + h * L
                        vx = pts[pl.ds(ib, L)]
                        vy = pts[pl.ds(N + ib, L)]
                        vz = pts[pl.ds(2 * N + ib, L)]
                        dx = vx - qx
                        dy = vy - qy
                        dz = vz - qz
                        d2 = dx * dx + dy * dy + dz * dz
                        m = d2 < RADIUS2
                        plsc.store_compressed(idxbuf.at[pl.ds(cnt1, L)],
                                              iota + ib, mask=m)
                        cnt1 = cnt1 + plsc.all_reduce_population_count(m)[0]
                    return ch + 1, cnt1

                _, cnt = lax.while_loop(cond, body,
                                        (jnp.int32(0), jnp.int32(0)))

                first = plsc.load_gather(idxbuf, [zeros16])
                sel0 = jnp.where(iota < cnt, idxbuf[pl.ds(0, L)], first)
                sel1 = jnp.where(iota + L < cnt, idxbuf[pl.ds(L, L)], first)
                idx_out[pl.ds(s * NSMP, L)] = sel0
                idx_out[pl.ds(s * NSMP + L, L)] = sel1
                for ci, qv in enumerate((qx, qy, qz)):
                    g0 = plsc.load_gather(pts, [sel0 + ci * N]) - qv
                    g1 = plsc.load_gather(pts, [sel1 + ci * N]) - qv
                    gx_out[pl.ds((ci * SQ + s) * NSMP, L)] = g0
                    gx_out[pl.ds((ci * SQ + s) * NSMP + L, L)] = g1
                return carry

            lax.fori_loop(0, SQ, per_query, 0)
            pltpu.sync_copy(idx_out,
                            idx_sh.at[pl.ds((lb * S + q0) * NSMP, SQ * NSMP)])
            for c in range(3):
                pltpu.sync_copy(
                    gx_out.at[pl.ds(c * SQ * NSMP, SQ * NSMP)],
                    out_hbm.at[b * OC + c, pl.ds(q0 * NSMP, SQ * NSMP)])

        pl.run_scoped(
            phase_a,
            pltpu.VMEM((3 * N,), jnp.float32),
            pltpu.VMEM((SQ * NSMP,), jnp.int32),
            pltpu.VMEM((3 * SQ * NSMP,), jnp.float32),
        )

        plsc.subcore_barrier()

        u0 = wid * CPT
        b2 = u0 // C
        c0 = u0 % C
        lb2 = b2 % BPC

        def phase_b(feats, outb):
            pltpu.sync_copy(
                feat_hbm.at[pl.ds(u0, CPT), :], feats)

            def per_chunk(qc, carry):
                pltpu.sync_copy(
                    idx_sh.at[pl.ds((lb2 * S + qc * CH) * NSMP, CH * NSMP)],
                    idxc)

                def per_vreg(k, c2):
                    ids = idxc[pl.ds(k * L, L)]
                    for j in range(CPT):
                        outb[j, pl.ds(k * L, L)] = plsc.load_gather(
                            feats.at[j], [ids])
                    return c2

                lax.fori_loop(0, KPC, per_vreg, 0)
                pltpu.sync_copy(
                    outb,
                    out_hbm.at[pl.ds(b2 * OC + 3 + c0, CPT),
                               pl.ds(qc * CH * NSMP, CH * NSMP)])
                return carry

            lax.fori_loop(0, NQC, per_chunk, 0)

        pl.run_scoped(
            phase_b,
            pltpu.VMEM((CPT, N), jnp.float32),
            pltpu.VMEM((CPT, CH * NSMP), jnp.float32),
        )

    return fused


def kernel(xyz, new_xyz, features):
    B, N, _ = xyz.shape
    S = new_xyz.shape[1]
    C = features.shape[1]
    xyz_t = jnp.transpose(xyz, (0, 2, 1)).reshape(-1)
    q_t = jnp.transpose(new_xyz, (0, 2, 1)).reshape(-1)
    feat2d = features.reshape(B * C, N)
    out = _make_fused(B, N, S, C)(xyz_t, q_t, feat2d)
    return out.reshape(B, C + 3, S, NSMP)


# speedup vs baseline: 1.4686x; 1.4686x over previous
import functools

import jax
import jax.numpy as jnp
from jax import lax
from jax.experimental import pallas as pl
from jax.experimental.pallas import tpu as pltpu
from jax.experimental.pallas import tpu_sc as plsc

RADIUS2 = 0.2 * 0.2
NSMP = 32
L = 16
NW = 32


@functools.lru_cache(maxsize=None)
def _make_fused(B, N, S, C):
    NB = NW // B
    SQ = S // NB
    NCH = N // L
    BPC = B // 2
    CPT = (B * C) // NW
    CH = 128
    NQC = S // CH
    OC = C + 3
    mesh = plsc.VectorSubcoreMesh(core_axis_name="c", subcore_axis_name="s")

    @functools.partial(
        pl.kernel,
        out_type=jax.ShapeDtypeStruct((B, OC, S, NSMP), jnp.float32),
        mesh=mesh,
        compiler_params=pltpu.CompilerParams(needs_layout_passes=False,
                                             use_tc_tiling_on_sc=False),
        scratch_types=[
            pltpu.VMEM((3 * SQ,), jnp.float32),
            pltpu.VMEM((128,), jnp.int32),
            pltpu.VMEM((CH, NSMP), jnp.int32),
            pltpu.VMEM_SHARED((BPC * S, NSMP), jnp.int32),
        ],
    )
    def fused(xyz_hbm, q_hbm, feat_hbm, out_hbm, qrs, idxbuf, idxc, idx_sh):
        wid = lax.axis_index("c") * 16 + lax.axis_index("s")

        b = wid // NB
        q0 = (wid % NB) * SQ
        lb = b % BPC

        def phase_a(pts, idx_out, gx_out):
            for c in range(3):
                pltpu.sync_copy(xyz_hbm.at[b, c], pts.at[pl.ds(c * N, N)])
                pltpu.sync_copy(q_hbm.at[b, c, pl.ds(q0, SQ)],
                                qrs.at[pl.ds(c * SQ, SQ)])
            iota = lax.broadcasted_iota(jnp.int32, (L,), 0)
            zeros16 = jnp.zeros((L,), jnp.int32)

            def per_query(s, carry):
                idxbuf[pl.ds(0, L)] = zeros16
                qx = plsc.load_gather(qrs, [jnp.full((L,), s, jnp.int32)])
                qy = plsc.load_gather(qrs, [jnp.full((L,), SQ + s, jnp.int32)])
                qz = plsc.load_gather(qrs,
                                      [jnp.full((L,), 2 * SQ + s, jnp.int32)])

                def cond(cr):
                    ch, cnt = cr
                    return jnp.logical_and(cnt < NSMP, ch < NCH // 4)

                def body(cr):
                    ch, cnt = cr
                    i0 = ch * (4 * L)
                    coords = [(pts[pl.ds(i0 + h * L, L)],
                               pts[pl.ds(N + i0 + h * L, L)],
                               pts[pl.ds(2 * N + i0 + h * L, L)])
                              for h in range(4)]
                    ms = []
                    for vx, vy, vz in coords:
                        dx = vx - qx
                        dy = vy - qy
                        dz = vz - qz
                        ms.append(dx * dx + dy * dy + dz * dz < RADIUS2)
                    pcs = [plsc.all_reduce_population_count(m) for m in ms]
                    cnt1 = cnt
                    for h, m in enumerate(ms):
                        plsc.store_compressed(idxbuf.at[pl.ds(cnt1, L)],
                                              iota + (i0 + h * L), mask=m)
                        cnt1 = cnt1 + pcs[h][0]
                    return ch + 1, cnt1

                _, cnt = lax.while_loop(cond, body,
                                        (jnp.int32(0), jnp.int32(0)))

                first = plsc.load_gather(idxbuf, [zeros16])
                sel0 = jnp.where(iota < cnt, idxbuf[pl.ds(0, L)], first)
                sel1 = jnp.where(iota + L < cnt, idxbuf[pl.ds(L, L)], first)
                idx_out[s, pl.ds(0, L)] = sel0
                idx_out[s, pl.ds(L, L)] = sel1
                for ci, qv in enumerate((qx, qy, qz)):
                    g0 = plsc.load_gather(pts, [sel0 + ci * N]) - qv
                    g1 = plsc.load_gather(pts, [sel1 + ci * N]) - qv
                    gx_out[ci, s, pl.ds(0, L)] = g0
                    gx_out[ci, s, pl.ds(L, L)] = g1
                return carry

            lax.fori_loop(0, SQ, per_query, 0)
            pltpu.sync_copy(idx_out, idx_sh.at[pl.ds(lb * S + q0, SQ), :])
            for c in range(3):
                pltpu.sync_copy(gx_out.at[c], out_hbm.at[b, c, pl.ds(q0, SQ)])

        pl.run_scoped(
            phase_a,
            pltpu.VMEM((3 * N,), jnp.float32),
            pltpu.VMEM((SQ, NSMP), jnp.int32),
            pltpu.VMEM((3, SQ, NSMP), jnp.float32),
        )

        plsc.subcore_barrier()

        u0 = wid * CPT
        b2 = u0 // C
        c0 = u0 % C
        lb2 = b2 % BPC

        def phase_b(feats, outb):
            pltpu.sync_copy(feat_hbm.at[b2, pl.ds(c0, CPT), :], feats)

            def per_chunk(qc, carry):
                pltpu.sync_copy(idx_sh.at[pl.ds(lb2 * S + qc * CH, CH), :],
                                idxc)

                def per_row(r, c2):
                    ids0 = idxc[r, pl.ds(0, L)]
                    ids1 = idxc[r, pl.ds(L, L)]
                    for j in range(CPT):
                        outb[j, r, pl.ds(0, L)] = plsc.load_gather(
                            feats.at[j], [ids0])
                        outb[j, r, pl.ds(L, L)] = plsc.load_gather(
                            feats.at[j], [ids1])
                    return c2

                lax.fori_loop(0, CH, per_row, 0)
                pltpu.sync_copy(
                    outb,
                    out_hbm.at[b2, pl.ds(3 + c0, CPT), pl.ds(qc * CH, CH)])
                return carry

            lax.fori_loop(0, NQC, per_chunk, 0)

        pl.run_scoped(
            phase_b,
            pltpu.VMEM((CPT, N), jnp.float32),
            pltpu.VMEM((CPT, CH, NSMP), jnp.float32),
        )

    return fused


def kernel(xyz, new_xyz, features):
    B, N, _ = xyz.shape
    S = new_xyz.shape[1]
    C = features.shape[1]
    xyz_t = jnp.transpose(xyz, (0, 2, 1))
    q_t = jnp.transpose(new_xyz, (0, 2, 1))
    return _make_fused(B, N, S, C)(xyz_t, q_t, features)


# speedup vs baseline: 2.8305x; 1.9274x over previous
import functools

import jax
import jax.numpy as jnp
from jax import lax
from jax.experimental import pallas as pl
from jax.experimental.pallas import tpu as pltpu
from jax.experimental.pallas import tpu_sc as plsc

RADIUS2 = 0.2 * 0.2
NSMP = 32
L = 16
NW = 32


@functools.lru_cache(maxsize=None)
def _make_fused(B, N, S, C):
    NB = NW // B
    SQ = S // NB
    NCH = N // L
    BPC = B // 2
    CPT = (B * C) // NW
    CH = 128
    NQC = S // CH
    OC = C + 3
    mesh = plsc.VectorSubcoreMesh(core_axis_name="c", subcore_axis_name="s")

    @functools.partial(
        pl.kernel,
        out_type=jax.ShapeDtypeStruct((B, OC, NSMP, S), jnp.float32),
        mesh=mesh,
        compiler_params=pltpu.CompilerParams(needs_layout_passes=False,
                                             use_tc_tiling_on_sc=True),
        scratch_types=[
            pltpu.VMEM((3 * SQ,), jnp.float32),
            pltpu.VMEM((256,), jnp.int32),
            pltpu.VMEM((NSMP, CH), jnp.int32),
            pltpu.VMEM_SHARED((BPC * NB, NSMP, SQ), jnp.int32),
            pltpu.SemaphoreType.DMA,
            pltpu.SemaphoreType.DMA,
        ],
    )
    def fused(xyz_hbm, q_hbm, feat_hbm, out_hbm, qrs, idxbuf, idxc, idx_sh,
              sem0, sem1):
        wid = lax.axis_index("c") * 16 + lax.axis_index("s")

        b = wid // NB
        q0 = (wid % NB) * SQ
        lb = b % BPC

        def phase_a(pts, stage, idx_out, gx_out):
            iota = lax.broadcasted_iota(jnp.int32, (L,), 0)
            zeros16 = jnp.zeros((L,), jnp.int32)
            pltpu.sync_copy(xyz_hbm.at[pl.ds(b * N * 3, N * 3)], stage)
            pltpu.sync_copy(q_hbm.at[pl.ds((b * S + q0) * 3, SQ * 3)],
                            qrs)
            iota3 = iota * 3

            def tr(g, carry):
                for c in range(3):
                    v = plsc.load_gather(stage, [iota3 + (g * (3 * L) + c)])
                    pts[pl.ds(c * N + g * L, L)] = v
                return carry

            lax.fori_loop(0, N // L, tr, 0)

            def per_query(s, carry):
                idxbuf[pl.ds(0, L)] = zeros16
                qx = plsc.load_gather(qrs, [jnp.full((L,), 3 * s, jnp.int32)])
                qy = plsc.load_gather(qrs,
                                      [jnp.full((L,), 3 * s + 1, jnp.int32)])
                qz = plsc.load_gather(qrs,
                                      [jnp.full((L,), 3 * s + 2, jnp.int32)])

                def cond(cr):
                    ch, cnt = cr
                    return jnp.logical_and(cnt < NSMP, ch < NCH // 8)

                def body(cr):
                    ch, cnt = cr
                    i0 = ch * (8 * L)
                    coords = [(pts[pl.ds(i0 + h * L, L)],
                               pts[pl.ds(N + i0 + h * L, L)],
                               pts[pl.ds(2 * N + i0 + h * L, L)])
                              for h in range(8)]
                    ms = []
                    for vx, vy, vz in coords:
                        dx = vx - qx
                        dy = vy - qy
                        dz = vz - qz
                        ms.append(dx * dx + dy * dy + dz * dz < RADIUS2)
                    pcs = [plsc.all_reduce_population_count(m) for m in ms]
                    cnt1 = cnt
                    for h, m in enumerate(ms):
                        plsc.store_compressed(idxbuf.at[pl.ds(cnt1, L)],
                                              iota + (i0 + h * L), mask=m)
                        cnt1 = cnt1 + pcs[h][0]
                    return ch + 1, cnt1

                _, cnt = lax.while_loop(cond, body,
                                        (jnp.int32(0), jnp.int32(0)))

                first = plsc.load_gather(idxbuf, [zeros16])
                sel0 = jnp.where(iota < cnt, idxbuf[pl.ds(0, L)], first)
                sel1 = jnp.where(iota + L < cnt, idxbuf[pl.ds(L, L)], first)
                fs = jnp.full((L,), s, jnp.int32)
                plsc.store_scatter(idx_out, [iota, fs], sel0)
                plsc.store_scatter(idx_out, [iota + L, fs], sel1)
                for ci, qv in enumerate((qx, qy, qz)):
                    fc = jnp.full((L,), ci, jnp.int32)
                    g0 = plsc.load_gather(pts, [sel0 + ci * N]) - qv
                    g1 = plsc.load_gather(pts, [sel1 + ci * N]) - qv
                    plsc.store_scatter(gx_out, [fc, iota, fs], g0)
                    plsc.store_scatter(gx_out, [fc, iota + L, fs], g1)
                return carry

            lax.fori_loop(0, SQ, per_query, 0)
            pltpu.sync_copy(idx_out, idx_sh.at[lb * NB + (wid % NB)])
            for c in range(3):
                pltpu.sync_copy(gx_out.at[c],
                                out_hbm.at[b, c, :, pl.ds(q0, SQ)])

        pl.run_scoped(
            phase_a,
            pltpu.VMEM((3 * N,), jnp.float32),
            pltpu.VMEM((3 * N,), jnp.float32),
            pltpu.VMEM((NSMP, SQ), jnp.int32),
            pltpu.VMEM((3, NSMP, SQ), jnp.float32),
        )

        plsc.subcore_barrier()

        u0 = wid * CPT
        b2 = u0 // C
        c0 = u0 % C
        lb2 = b2 % BPC

        HS = NSMP // 2

        def phase_b(feats, outb0, outb1):
            pltpu.sync_copy(feat_hbm.at[pl.ds((b2 * C + c0) * N, CPT * N)],
                            feats)
            cpb = SQ // CH
            outbs = (outb0, outb1)
            sems = (sem0, sem1)

            def load_idx(qc):
                pltpu.sync_copy(
                    idx_sh.at[lb2 * NB + qc // cpb, :,
                              pl.ds((qc % cpb) * CH, CH)],
                    idxc)

            def gather_half(h, buf):
                def per_slot(k, c2):
                    for g in range(CH // L):
                        ids = idxc[h * HS + k, pl.ds(g * L, L)]
                        for j in range(CPT):
                            buf[j, k, pl.ds(g * L, L)] = plsc.load_gather(
                                feats.at[pl.ds(j * N, N)], [ids])
                    return c2

                lax.fori_loop(0, HS, per_slot, 0)

            def out_dst(qc, h):
                return out_hbm.at[b2, pl.ds(3 + c0, CPT),
                                  pl.ds(h * HS, HS), pl.ds(qc * CH, CH)]

            load_idx(0)
            for h in range(2):
                gather_half(h, outbs[h])
                pltpu.async_copy(outbs[h], out_dst(0, h), sems[h])

            def per_chunk(qc, carry):
                load_idx(qc)
                for h in range(2):
                    pltpu.make_async_copy(outbs[h], out_dst(qc - 1, h),
                                          sems[h]).wait()
                    gather_half(h, outbs[h])
                    pltpu.async_copy(outbs[h], out_dst(qc, h), sems[h])
                return carry

            lax.fori_loop(1, NQC, per_chunk, 0)
            for h in range(2):
                pltpu.make_async_copy(outbs[h], out_dst(NQC - 1, h),
                                      sems[h]).wait()

        pl.run_scoped(
            phase_b,
            pltpu.VMEM((CPT * N,), jnp.float32),
            pltpu.VMEM((CPT, HS, CH), jnp.float32),
            pltpu.VMEM((CPT, HS, CH), jnp.float32),
        )

    return fused


def kernel(xyz, new_xyz, features):
    B, N, _ = xyz.shape
    S = new_xyz.shape[1]
    C = features.shape[1]
    out_t = _make_fused(B, N, S, C)(xyz.reshape(-1), new_xyz.reshape(-1),
                                    features.reshape(-1))
    return jnp.swapaxes(out_t, 2, 3)
